# baseline (speedup 1.0000x reference)
import functools

import jax
import jax.numpy as jnp
from jax import lax
from jax.experimental import pallas as pl
from jax.experimental.pallas import tpu as pltpu
from jax.experimental.pallas import tpu_sc as plsc

N = 10000
E = 320000
IN_DIM = 128
H = 32
OUT_DIM = 128
R = 4

GROUP = 128
NW = 32
G = 2560
EP = G * GROUP
GPT = G // NW
GPT0 = 56
GPT1 = 104
NP = 10240
ROWS_PER_TILE = NP // 16
ZCH = 128
NT = N * R


def _edge_prep_body(et_ref, src_ref, dst_ref, typ_ref, lam_ref, beta_ref,
                    w_ref, ia_ref, ib_ref):
    lam = lam_ref[0, 0]
    beta = beta_ref[0, 0]
    valid = lax.broadcasted_iota(jnp.int32, (G, GROUP), 0) < (E // GROUP)
    w = lam * jnp.exp(-beta * jnp.abs(et_ref[...]))
    w_ref[...] = jnp.where(valid, w, 0.0)
    typ = typ_ref[...]
    ia_ref[...] = src_ref[...] * R + typ
    ib_ref[...] = dst_ref[...] * R + typ


def _edge_prep(et2, src2, dst2, typ2, lam, beta):
    return pl.pallas_call(
        _edge_prep_body,
        out_shape=(
            jax.ShapeDtypeStruct((G, GROUP), jnp.float32),
            jax.ShapeDtypeStruct((G, GROUP), jnp.int32),
            jax.ShapeDtypeStruct((G, GROUP), jnp.int32),
        ),
    )(et2, src2, dst2, typ2, lam, beta)


BN = 2000


def _embed_body(x_ref, wf_ref, bf_ref, ws_ref, wd_ref, brf_ref,
                h0_ref, a_ref, b_ref):
    h0 = jnp.dot(x_ref[...], wf_ref[...], preferred_element_type=jnp.float32)
    h0 = h0 + bf_ref[...]
    h0_ref[...] = h0
    a_ref[...] = jnp.dot(h0, ws_ref[...], preferred_element_type=jnp.float32)
    b_ref[...] = jnp.dot(h0, wd_ref[...],
                         preferred_element_type=jnp.float32) + brf_ref[...]


def _embed(x, wf, bf, ws, wd, brf):
    grid = N // BN
    return pl.pallas_call(
        _embed_body,
        grid=(grid,),
        in_specs=[
            pl.BlockSpec((BN, IN_DIM), lambda i: (i, 0)),
            pl.BlockSpec((IN_DIM, H), lambda i: (0, 0)),
            pl.BlockSpec((1, H), lambda i: (0, 0)),
            pl.BlockSpec((H, R * H), lambda i: (0, 0)),
            pl.BlockSpec((H, R * H), lambda i: (0, 0)),
            pl.BlockSpec((1, R * H), lambda i: (0, 0)),
        ],
        out_specs=(
            pl.BlockSpec((BN, H), lambda i: (i, 0)),
            pl.BlockSpec((BN, R * H), lambda i: (i, 0)),
            pl.BlockSpec((BN, R * H), lambda i: (i, 0)),
        ),
        out_shape=(
            jax.ShapeDtypeStruct((N, H), jnp.float32),
            jax.ShapeDtypeStruct((N, R * H), jnp.float32),
            jax.ShapeDtypeStruct((N, R * H), jnp.float32),
        ),
    )(x, wf, bf, ws, wd, brf)


def _mid_body(p_ref, ws_ref, wd_ref, brf_ref, h_ref, a_ref, b_ref):
    h = p_ref[0] + p_ref[1]
    h_ref[...] = h
    a_ref[...] = jnp.dot(h, ws_ref[...], preferred_element_type=jnp.float32)
    b_ref[...] = jnp.dot(h, wd_ref[...],
                         preferred_element_type=jnp.float32) + brf_ref[...]


def _mid(p, ws, wd, brf):
    grid = N // BN
    return pl.pallas_call(
        _mid_body,
        grid=(grid,),
        in_specs=[
            pl.BlockSpec((2, BN, H), lambda i: (0, i, 0)),
            pl.BlockSpec((H, R * H), lambda i: (0, 0)),
            pl.BlockSpec((H, R * H), lambda i: (0, 0)),
            pl.BlockSpec((1, R * H), lambda i: (0, 0)),
        ],
        out_specs=(
            pl.BlockSpec((BN, H), lambda i: (i, 0)),
            pl.BlockSpec((BN, R * H), lambda i: (i, 0)),
            pl.BlockSpec((BN, R * H), lambda i: (i, 0)),
        ),
        out_shape=(
            jax.ShapeDtypeStruct((N, H), jnp.float32),
            jax.ShapeDtypeStruct((N, R * H), jnp.float32),
            jax.ShapeDtypeStruct((N, R * H), jnp.float32),
        ),
    )(p, ws, wd, brf)


def _lrelu(t):
    return jnp.where(t > 0, t, 0.01 * t)


def _final_body(p2_ref, h1_ref, h0_ref, wo0_ref, bo0_ref, wo1_ref, bo1_ref,
                wo2_ref, bo2_ref, out_ref):
    h2 = p2_ref[0] + p2_ref[1]
    t2 = jnp.dot(h2, wo2_ref[...], preferred_element_type=jnp.float32) + bo2_ref[...]
    t1 = jnp.dot(h1_ref[...], wo1_ref[...],
                 preferred_element_type=jnp.float32) + bo1_ref[...]
    t0 = jnp.dot(h0_ref[...], wo0_ref[...],
                 preferred_element_type=jnp.float32) + bo0_ref[...]
    out_ref[...] = _lrelu(t2) + _lrelu(t1) + _lrelu(t0)


def _final(p2, h1, h0, wo0, bo0, wo1, bo1, wo2, bo2):
    grid = N // BN
    wspec = pl.BlockSpec((H, OUT_DIM), lambda i: (0, 0))
    bspec = pl.BlockSpec((1, OUT_DIM), lambda i: (0, 0))
    return pl.pallas_call(
        _final_body,
        grid=(grid,),
        in_specs=[
            pl.BlockSpec((2, BN, H), lambda i: (0, i, 0)),
            pl.BlockSpec((BN, H), lambda i: (i, 0)),
            pl.BlockSpec((BN, H), lambda i: (i, 0)),
            wspec, bspec, wspec, bspec, wspec, bspec,
        ],
        out_specs=pl.BlockSpec((BN, OUT_DIM), lambda i: (i, 0)),
        out_shape=jax.ShapeDtypeStruct((N, OUT_DIM), jnp.float32),
    )(p2, h1, h0, wo0, bo0, wo1, bo1, wo2, bo2)


def _sc_layer_body(a_hbm, b_hbm, idxa_hbm, idxb_hbm, dst_hbm, w_hbm, out_hbm,
                   idxa_v, idxb_v, dst_v, w_v, a0, a1, b0, b1, o0, o1,
                   zbuf, acc, sga0, sga1, sgb0, sgb1, ss0, ss1):
    cid = lax.axis_index("c")
    sid = lax.axis_index("s")
    ng = jnp.where(cid == 0, GPT0, GPT1)
    gbase = jnp.where(cid == 0, sid * GPT0, 16 * GPT0 + sid * GPT1)

    pltpu.sync_copy(idxa_hbm.at[pl.ds(gbase, GPT1)], idxa_v)
    pltpu.sync_copy(idxb_hbm.at[pl.ds(gbase, GPT1)], idxb_v)
    pltpu.sync_copy(dst_hbm.at[pl.ds(gbase, GPT1)], dst_v)
    pltpu.sync_copy(w_hbm.at[pl.ds(gbase, GPT1)], w_v)

    def zb(i, c):
        zbuf[i, 0:16] = jnp.zeros((16,), jnp.float32)
        zbuf[i, 16:32] = jnp.zeros((16,), jnp.float32)
        return c

    lax.fori_loop(0, ZCH, zb, 0)
    rbase = sid * ROWS_PER_TILE
    for j in range(ROWS_PER_TILE // ZCH):
        pltpu.sync_copy(zbuf.at[pl.ds(0, ZCH)],
                        acc.at[pl.ds(rbase + j * ZCH, ZCH)])
    plsc.subcore_barrier()

    abufs = (a0, a1)
    bbufs = (b0, b1)
    obufs = (o0, o1)
    sgas = (sga0, sga1)
    sgbs = (sgb0, sgb1)
    sss = (ss0, ss1)

    for p in range(2):
        pltpu.async_copy(a_hbm.at[idxa_v.at[p]], abufs[p], sgas[p])
        pltpu.async_copy(b_hbm.at[idxb_v.at[p]], bbufs[p], sgbs[p])

    def pair(k2, c):
        for p in range(2):
            k = k2 * 2 + p
            ab, bb, ob = abufs[p], bbufs[p], obufs[p]
            pltpu.make_async_copy(a_hbm.at[idxa_v.at[k]], ab, sgas[p]).wait()
            pltpu.make_async_copy(b_hbm.at[idxb_v.at[k]], bb, sgbs[p]).wait()

            @pl.when(k2 > 0)
            def _():
                pltpu.make_async_copy(ob, acc.at[dst_v.at[k]], sss[p]).wait()

            def ebody(j, cc):
                wv16 = w_v[k, pl.ds(j * 16, 16)]
                for ll in range(16):
                    i = j * 16 + ll
                    wv = wv16[ll]
                    ob[i, 0:16] = (ab[i, 0:16] + bb[i, 0:16]) * wv
                    ob[i, 16:32] = (ab[i, 16:32] + bb[i, 16:32]) * wv
                return cc

            lax.fori_loop(0, GROUP // 16, ebody, 0)

            @pl.when(k + 2 < ng)
            def _():
                pltpu.async_copy(a_hbm.at[idxa_v.at[k + 2]], ab, sgas[p])
                pltpu.async_copy(b_hbm.at[idxb_v.at[k + 2]], bb, sgbs[p])

            pltpu.async_copy(ob, acc.at[dst_v.at[k]], sss[p], add=True)
        return c

    lax.fori_loop(0, ng // 2, pair, 0)
    for p in range(2):
        pltpu.make_async_copy(obufs[p], acc.at[dst_v.at[ng - 2 + p]],
                              sss[p]).wait()
    plsc.subcore_barrier()

    pltpu.sync_copy(acc.at[pl.ds(rbase, ROWS_PER_TILE)],
                    out_hbm.at[cid, pl.ds(rbase, ROWS_PER_TILE)])


def _sc_layer(a2d, b2d, idxa2, idxb2, dst2, w2):
    mesh = plsc.VectorSubcoreMesh(core_axis_name="c", subcore_axis_name="s")
    kern = functools.partial(
        pl.kernel,
        mesh=mesh,
        compiler_params=pltpu.CompilerParams(use_tc_tiling_on_sc=False),
        out_type=jax.ShapeDtypeStruct((2, NP, H), jnp.float32),
        scratch_types=[
            pltpu.VMEM((GPT1, GROUP), jnp.int32),
            pltpu.VMEM((GPT1, GROUP), jnp.int32),
            pltpu.VMEM((GPT1, GROUP), jnp.int32),
            pltpu.VMEM((GPT1, GROUP), jnp.float32),
            pltpu.VMEM((GROUP, H), jnp.float32),
            pltpu.VMEM((GROUP, H), jnp.float32),
            pltpu.VMEM((GROUP, H), jnp.float32),
            pltpu.VMEM((GROUP, H), jnp.float32),
            pltpu.VMEM((GROUP, H), jnp.float32),
            pltpu.VMEM((GROUP, H), jnp.float32),
            pltpu.VMEM((ZCH, H), jnp.float32),
            pltpu.VMEM_SHARED((NP, H), jnp.float32),
            pltpu.SemaphoreType.DMA,
            pltpu.SemaphoreType.DMA,
            pltpu.SemaphoreType.DMA,
            pltpu.SemaphoreType.DMA,
            pltpu.SemaphoreType.DMA,
            pltpu.SemaphoreType.DMA,
        ],
    )(_sc_layer_body)
    return kern(a2d, b2d, idxa2, idxb2, dst2, w2)


def kernel(x, edge_time, lambda_sym, beta, Wf, bf, Wr1, br1, Wr2, br2,
           Wo0, bo0, Wo1, bo1, Wo2, bo2, edge_index, edge_type):
    ws1 = jnp.transpose(Wr1[:, :H, :], (1, 0, 2)).reshape(H, R * H)
    wd1 = jnp.transpose(Wr1[:, H:, :], (1, 0, 2)).reshape(H, R * H)
    ws2 = jnp.transpose(Wr2[:, :H, :], (1, 0, 2)).reshape(H, R * H)
    wd2 = jnp.transpose(Wr2[:, H:, :], (1, 0, 2)).reshape(H, R * H)
    brf1 = br1.reshape(1, R * H)
    brf2 = br2.reshape(1, R * H)
    bfr = bf.reshape(1, H)
    bo0r = bo0.reshape(1, OUT_DIM)
    bo1r = bo1.reshape(1, OUT_DIM)
    bo2r = bo2.reshape(1, OUT_DIM)

    pad = EP - E
    et2 = jnp.pad(edge_time, (0, pad)).reshape(G, GROUP)
    src2 = jnp.pad(edge_index[0], (0, pad)).reshape(G, GROUP)
    dst2 = jnp.pad(edge_index[1], (0, pad)).reshape(G, GROUP)
    typ2 = jnp.pad(edge_type, (0, pad)).reshape(G, GROUP)

    w2, idxa2, idxb2 = _edge_prep(et2, src2, dst2, typ2, lambda_sym, beta)

    h0, a1, b1 = _embed(x, Wf, bfr, ws1, wd1, brf1)
    p1 = _sc_layer(a1.reshape(NT, H), b1.reshape(NT, H),
                   idxa2, idxb2, dst2, w2)[:, :N, :]
    h1, a2, b2 = _mid(p1, ws2, wd2, brf2)
    p2 = _sc_layer(a2.reshape(NT, H), b2.reshape(NT, H),
                   idxa2, idxb2, dst2, w2)[:, :N, :]
    out = _final(p2, h1, h0, Wo0, bo0r, Wo1, bo1r, Wo2, bo2r)
    return out


# speedup vs baseline: 1.1058x; 1.1058x over previous
import functools

import jax
import jax.numpy as jnp
from jax import lax
from jax.experimental import pallas as pl
from jax.experimental.pallas import tpu as pltpu
from jax.experimental.pallas import tpu_sc as plsc

N = 10000
E = 320000
IN_DIM = 128
H = 32
OUT_DIM = 128
R = 4

GROUP = 128
NW = 32
G = 2560
EP = G * GROUP
GPT = G // NW
GPT0 = 104
GPT1 = 56
GPTMAX = max(GPT0, GPT1)
NP = 10240
ROWS_PER_TILE = NP // 16
ZCH = 128
NT = N * R


def _edge_prep_body(et_ref, src_ref, dst_ref, typ_ref, lam_ref, beta_ref,
                    w_ref, ia_ref, ib_ref):
    lam = lam_ref[0, 0]
    beta = beta_ref[0, 0]
    valid = lax.broadcasted_iota(jnp.int32, (G, GROUP), 0) < (E // GROUP)
    w = lam * jnp.exp(-beta * jnp.abs(et_ref[...]))
    w_ref[...] = jnp.where(valid, w, 0.0)
    typ = typ_ref[...]
    ia_ref[...] = src_ref[...] * R + typ
    ib_ref[...] = dst_ref[...] * R + typ


def _edge_prep(et2, src2, dst2, typ2, lam, beta):
    return pl.pallas_call(
        _edge_prep_body,
        out_shape=(
            jax.ShapeDtypeStruct((G, GROUP), jnp.float32),
            jax.ShapeDtypeStruct((G, GROUP), jnp.int32),
            jax.ShapeDtypeStruct((G, GROUP), jnp.int32),
        ),
    )(et2, src2, dst2, typ2, lam, beta)


BN = 2000


def _embed_body(x_ref, wf_ref, bf_ref, ws_ref, wd_ref, brf_ref,
                h0_ref, a_ref, b_ref):
    h0 = jnp.dot(x_ref[...], wf_ref[...], preferred_element_type=jnp.float32)
    h0 = h0 + bf_ref[...]
    h0_ref[...] = h0
    a_ref[...] = jnp.dot(h0, ws_ref[...], preferred_element_type=jnp.float32)
    b_ref[...] = jnp.dot(h0, wd_ref[...],
                         preferred_element_type=jnp.float32) + brf_ref[...]


def _embed(x, wf, bf, ws, wd, brf):
    grid = N // BN
    return pl.pallas_call(
        _embed_body,
        grid=(grid,),
        in_specs=[
            pl.BlockSpec((BN, IN_DIM), lambda i: (i, 0)),
            pl.BlockSpec((IN_DIM, H), lambda i: (0, 0)),
            pl.BlockSpec((1, H), lambda i: (0, 0)),
            pl.BlockSpec((H, R * H), lambda i: (0, 0)),
            pl.BlockSpec((H, R * H), lambda i: (0, 0)),
            pl.BlockSpec((1, R * H), lambda i: (0, 0)),
        ],
        out_specs=(
            pl.BlockSpec((BN, H), lambda i: (i, 0)),
            pl.BlockSpec((BN, R * H), lambda i: (i, 0)),
            pl.BlockSpec((BN, R * H), lambda i: (i, 0)),
        ),
        out_shape=(
            jax.ShapeDtypeStruct((N, H), jnp.float32),
            jax.ShapeDtypeStruct((N, R * H), jnp.float32),
            jax.ShapeDtypeStruct((N, R * H), jnp.float32),
        ),
    )(x, wf, bf, ws, wd, brf)


def _mid_body(p_ref, ws_ref, wd_ref, brf_ref, h_ref, a_ref, b_ref):
    h = p_ref[0] + p_ref[1]
    h_ref[...] = h
    a_ref[...] = jnp.dot(h, ws_ref[...], preferred_element_type=jnp.float32)
    b_ref[...] = jnp.dot(h, wd_ref[...],
                         preferred_element_type=jnp.float32) + brf_ref[...]


def _mid(p, ws, wd, brf):
    grid = N // BN
    return pl.pallas_call(
        _mid_body,
        grid=(grid,),
        in_specs=[
            pl.BlockSpec((2, BN, H), lambda i: (0, i, 0)),
            pl.BlockSpec((H, R * H), lambda i: (0, 0)),
            pl.BlockSpec((H, R * H), lambda i: (0, 0)),
            pl.BlockSpec((1, R * H), lambda i: (0, 0)),
        ],
        out_specs=(
            pl.BlockSpec((BN, H), lambda i: (i, 0)),
            pl.BlockSpec((BN, R * H), lambda i: (i, 0)),
            pl.BlockSpec((BN, R * H), lambda i: (i, 0)),
        ),
        out_shape=(
            jax.ShapeDtypeStruct((N, H), jnp.float32),
            jax.ShapeDtypeStruct((N, R * H), jnp.float32),
            jax.ShapeDtypeStruct((N, R * H), jnp.float32),
        ),
    )(p, ws, wd, brf)


def _lrelu(t):
    return jnp.where(t > 0, t, 0.01 * t)


def _final_body(p2_ref, h1_ref, h0_ref, wo0_ref, bo0_ref, wo1_ref, bo1_ref,
                wo2_ref, bo2_ref, out_ref):
    h2 = p2_ref[0] + p2_ref[1]
    t2 = jnp.dot(h2, wo2_ref[...], preferred_element_type=jnp.float32) + bo2_ref[...]
    t1 = jnp.dot(h1_ref[...], wo1_ref[...],
                 preferred_element_type=jnp.float32) + bo1_ref[...]
    t0 = jnp.dot(h0_ref[...], wo0_ref[...],
                 preferred_element_type=jnp.float32) + bo0_ref[...]
    out_ref[...] = _lrelu(t2) + _lrelu(t1) + _lrelu(t0)


def _final(p2, h1, h0, wo0, bo0, wo1, bo1, wo2, bo2):
    grid = N // BN
    wspec = pl.BlockSpec((H, OUT_DIM), lambda i: (0, 0))
    bspec = pl.BlockSpec((1, OUT_DIM), lambda i: (0, 0))
    return pl.pallas_call(
        _final_body,
        grid=(grid,),
        in_specs=[
            pl.BlockSpec((2, BN, H), lambda i: (0, i, 0)),
            pl.BlockSpec((BN, H), lambda i: (i, 0)),
            pl.BlockSpec((BN, H), lambda i: (i, 0)),
            wspec, bspec, wspec, bspec, wspec, bspec,
        ],
        out_specs=pl.BlockSpec((BN, OUT_DIM), lambda i: (i, 0)),
        out_shape=jax.ShapeDtypeStruct((N, OUT_DIM), jnp.float32),
    )(p2, h1, h0, wo0, bo0, wo1, bo1, wo2, bo2)


def _sc_layer_body(a_hbm, b_hbm, idxa_hbm, idxb_hbm, dst_hbm, w_hbm, out_hbm,
                   idxa_v, idxb_v, dst_v, w_v, a0, a1, b0, b1, o0, o1,
                   zbuf, acc, sga0, sga1, sgb0, sgb1, ss0, ss1):
    cid = lax.axis_index("c")
    sid = lax.axis_index("s")
    ng = jnp.where(cid == 0, GPT0, GPT1)
    gbase = jnp.where(cid == 0, sid * GPT0, 16 * GPT0 + sid * GPT1)

    sbase = jnp.minimum(gbase, G - GPTMAX)
    off = gbase - sbase
    pltpu.sync_copy(idxa_hbm.at[pl.ds(sbase, GPTMAX)], idxa_v)
    pltpu.sync_copy(idxb_hbm.at[pl.ds(sbase, GPTMAX)], idxb_v)
    pltpu.sync_copy(dst_hbm.at[pl.ds(sbase, GPTMAX)], dst_v)
    pltpu.sync_copy(w_hbm.at[pl.ds(sbase, GPTMAX)], w_v)

    def zb(i, c):
        zbuf[i, 0:16] = jnp.zeros((16,), jnp.float32)
        zbuf[i, 16:32] = jnp.zeros((16,), jnp.float32)
        return c

    lax.fori_loop(0, ZCH, zb, 0)
    rbase = sid * ROWS_PER_TILE
    for j in range(ROWS_PER_TILE // ZCH):
        pltpu.sync_copy(zbuf.at[pl.ds(0, ZCH)],
                        acc.at[pl.ds(rbase + j * ZCH, ZCH)])
    plsc.subcore_barrier()

    abufs = (a0, a1)
    bbufs = (b0, b1)
    obufs = (o0, o1)
    sgas = (sga0, sga1)
    sgbs = (sgb0, sgb1)
    sss = (ss0, ss1)

    for p in range(2):
        pltpu.async_copy(a_hbm.at[idxa_v.at[off + p]], abufs[p], sgas[p])
        pltpu.async_copy(b_hbm.at[idxb_v.at[off + p]], bbufs[p], sgbs[p])

    def pair(k2, c):
        for p in range(2):
            k = k2 * 2 + p
            ab, bb, ob = abufs[p], bbufs[p], obufs[p]
            pltpu.make_async_copy(a_hbm.at[idxa_v.at[off + k]], ab, sgas[p]).wait()
            pltpu.make_async_copy(b_hbm.at[idxb_v.at[off + k]], bb, sgbs[p]).wait()

            @pl.when(k2 > 0)
            def _():
                pltpu.make_async_copy(ob, acc.at[dst_v.at[off + k]], sss[p]).wait()

            def ebody(j, cc):
                wv16 = w_v[off + k, pl.ds(j * 16, 16)]
                for ll in range(16):
                    i = j * 16 + ll
                    wv = wv16[ll]
                    ob[i, 0:16] = (ab[i, 0:16] + bb[i, 0:16]) * wv
                    ob[i, 16:32] = (ab[i, 16:32] + bb[i, 16:32]) * wv
                return cc

            lax.fori_loop(0, GROUP // 16, ebody, 0)

            @pl.when(k + 2 < ng)
            def _():
                pltpu.async_copy(a_hbm.at[idxa_v.at[off + k + 2]], ab, sgas[p])
                pltpu.async_copy(b_hbm.at[idxb_v.at[off + k + 2]], bb, sgbs[p])

            pltpu.async_copy(ob, acc.at[dst_v.at[off + k]], sss[p], add=True)
        return c

    lax.fori_loop(0, ng // 2, pair, 0)
    for p in range(2):
        pltpu.make_async_copy(obufs[p], acc.at[dst_v.at[off + ng - 2 + p]],
                              sss[p]).wait()
    plsc.subcore_barrier()

    pltpu.sync_copy(acc.at[pl.ds(rbase, ROWS_PER_TILE)],
                    out_hbm.at[cid, pl.ds(rbase, ROWS_PER_TILE)])


def _sc_layer(a2d, b2d, idxa2, idxb2, dst2, w2):
    mesh = plsc.VectorSubcoreMesh(core_axis_name="c", subcore_axis_name="s")
    kern = functools.partial(
        pl.kernel,
        mesh=mesh,
        compiler_params=pltpu.CompilerParams(use_tc_tiling_on_sc=False),
        out_type=jax.ShapeDtypeStruct((2, NP, H), jnp.float32),
        scratch_types=[
            pltpu.VMEM((GPTMAX, GROUP), jnp.int32),
            pltpu.VMEM((GPTMAX, GROUP), jnp.int32),
            pltpu.VMEM((GPTMAX, GROUP), jnp.int32),
            pltpu.VMEM((GPTMAX, GROUP), jnp.float32),
            pltpu.VMEM((GROUP, H), jnp.float32),
            pltpu.VMEM((GROUP, H), jnp.float32),
            pltpu.VMEM((GROUP, H), jnp.float32),
            pltpu.VMEM((GROUP, H), jnp.float32),
            pltpu.VMEM((GROUP, H), jnp.float32),
            pltpu.VMEM((GROUP, H), jnp.float32),
            pltpu.VMEM((ZCH, H), jnp.float32),
            pltpu.VMEM_SHARED((NP, H), jnp.float32),
            pltpu.SemaphoreType.DMA,
            pltpu.SemaphoreType.DMA,
            pltpu.SemaphoreType.DMA,
            pltpu.SemaphoreType.DMA,
            pltpu.SemaphoreType.DMA,
            pltpu.SemaphoreType.DMA,
        ],
    )(_sc_layer_body)
    return kern(a2d, b2d, idxa2, idxb2, dst2, w2)


def kernel(x, edge_time, lambda_sym, beta, Wf, bf, Wr1, br1, Wr2, br2,
           Wo0, bo0, Wo1, bo1, Wo2, bo2, edge_index, edge_type):
    ws1 = jnp.transpose(Wr1[:, :H, :], (1, 0, 2)).reshape(H, R * H)
    wd1 = jnp.transpose(Wr1[:, H:, :], (1, 0, 2)).reshape(H, R * H)
    ws2 = jnp.transpose(Wr2[:, :H, :], (1, 0, 2)).reshape(H, R * H)
    wd2 = jnp.transpose(Wr2[:, H:, :], (1, 0, 2)).reshape(H, R * H)
    brf1 = br1.reshape(1, R * H)
    brf2 = br2.reshape(1, R * H)
    bfr = bf.reshape(1, H)
    bo0r = bo0.reshape(1, OUT_DIM)
    bo1r = bo1.reshape(1, OUT_DIM)
    bo2r = bo2.reshape(1, OUT_DIM)

    pad = EP - E
    et2 = jnp.pad(edge_time, (0, pad)).reshape(G, GROUP)
    src2 = jnp.pad(edge_index[0], (0, pad)).reshape(G, GROUP)
    dst2 = jnp.pad(edge_index[1], (0, pad)).reshape(G, GROUP)
    typ2 = jnp.pad(edge_type, (0, pad)).reshape(G, GROUP)

    w2, idxa2, idxb2 = _edge_prep(et2, src2, dst2, typ2, lambda_sym, beta)

    h0, a1, b1 = _embed(x, Wf, bfr, ws1, wd1, brf1)
    p1 = _sc_layer(a1.reshape(NT, H), b1.reshape(NT, H),
                   idxa2, idxb2, dst2, w2)[:, :N, :]
    h1, a2, b2 = _mid(p1, ws2, wd2, brf2)
    p2 = _sc_layer(a2.reshape(NT, H), b2.reshape(NT, H),
                   idxa2, idxb2, dst2, w2)[:, :N, :]
    out = _final(p2, h1, h0, Wo0, bo0r, Wo1, bo1r, Wo2, bo2r)
    return out


# speedup vs baseline: 1.1107x; 1.0045x over previous
import functools

import jax
import jax.numpy as jnp
from jax import lax
from jax.experimental import pallas as pl
from jax.experimental.pallas import tpu as pltpu
from jax.experimental.pallas import tpu_sc as plsc

N = 10000
E = 320000
IN_DIM = 128
H = 32
OUT_DIM = 128
R = 4

GROUP = 128
NW = 32
G = 2560
EP = G * GROUP
GPT = G // NW
GPT0 = 120
GPT1 = 40
GPTMAX = max(GPT0, GPT1)
NP = 10240
ROWS_PER_TILE = NP // 16
ZCH = 128
NT = N * R


def _edge_prep_body(et_ref, src_ref, dst_ref, typ_ref, lam_ref, beta_ref,
                    w_ref, ia_ref, ib_ref):
    lam = lam_ref[0, 0]
    beta = beta_ref[0, 0]
    valid = lax.broadcasted_iota(jnp.int32, (G, GROUP), 0) < (E // GROUP)
    w = lam * jnp.exp(-beta * jnp.abs(et_ref[...]))
    w_ref[...] = jnp.where(valid, w, 0.0)
    typ = typ_ref[...]
    ia_ref[...] = src_ref[...] * R + typ
    ib_ref[...] = dst_ref[...] * R + typ


def _edge_prep(et2, src2, dst2, typ2, lam, beta):
    return pl.pallas_call(
        _edge_prep_body,
        out_shape=(
            jax.ShapeDtypeStruct((G, GROUP), jnp.float32),
            jax.ShapeDtypeStruct((G, GROUP), jnp.int32),
            jax.ShapeDtypeStruct((G, GROUP), jnp.int32),
        ),
    )(et2, src2, dst2, typ2, lam, beta)


BN = 2000


def _embed_body(x_ref, wf_ref, bf_ref, ws_ref, wd_ref, brf_ref,
                h0_ref, a_ref, b_ref):
    h0 = jnp.dot(x_ref[...], wf_ref[...], preferred_element_type=jnp.float32)
    h0 = h0 + bf_ref[...]
    h0_ref[...] = h0
    a_ref[...] = jnp.dot(h0, ws_ref[...], preferred_element_type=jnp.float32)
    b_ref[...] = jnp.dot(h0, wd_ref[...],
                         preferred_element_type=jnp.float32) + brf_ref[...]


def _embed(x, wf, bf, ws, wd, brf):
    grid = N // BN
    return pl.pallas_call(
        _embed_body,
        grid=(grid,),
        in_specs=[
            pl.BlockSpec((BN, IN_DIM), lambda i: (i, 0)),
            pl.BlockSpec((IN_DIM, H), lambda i: (0, 0)),
            pl.BlockSpec((1, H), lambda i: (0, 0)),
            pl.BlockSpec((H, R * H), lambda i: (0, 0)),
            pl.BlockSpec((H, R * H), lambda i: (0, 0)),
            pl.BlockSpec((1, R * H), lambda i: (0, 0)),
        ],
        out_specs=(
            pl.BlockSpec((BN, H), lambda i: (i, 0)),
            pl.BlockSpec((BN, R * H), lambda i: (i, 0)),
            pl.BlockSpec((BN, R * H), lambda i: (i, 0)),
        ),
        out_shape=(
            jax.ShapeDtypeStruct((N, H), jnp.float32),
            jax.ShapeDtypeStruct((N, R * H), jnp.float32),
            jax.ShapeDtypeStruct((N, R * H), jnp.float32),
        ),
    )(x, wf, bf, ws, wd, brf)


def _mid_body(p_ref, ws_ref, wd_ref, brf_ref, h_ref, a_ref, b_ref):
    h = p_ref[0] + p_ref[1]
    h_ref[...] = h
    a_ref[...] = jnp.dot(h, ws_ref[...], preferred_element_type=jnp.float32)
    b_ref[...] = jnp.dot(h, wd_ref[...],
                         preferred_element_type=jnp.float32) + brf_ref[...]


def _mid(p, ws, wd, brf):
    grid = N // BN
    return pl.pallas_call(
        _mid_body,
        grid=(grid,),
        in_specs=[
            pl.BlockSpec((2, BN, H), lambda i: (0, i, 0)),
            pl.BlockSpec((H, R * H), lambda i: (0, 0)),
            pl.BlockSpec((H, R * H), lambda i: (0, 0)),
            pl.BlockSpec((1, R * H), lambda i: (0, 0)),
        ],
        out_specs=(
            pl.BlockSpec((BN, H), lambda i: (i, 0)),
            pl.BlockSpec((BN, R * H), lambda i: (i, 0)),
            pl.BlockSpec((BN, R * H), lambda i: (i, 0)),
        ),
        out_shape=(
            jax.ShapeDtypeStruct((N, H), jnp.float32),
            jax.ShapeDtypeStruct((N, R * H), jnp.float32),
            jax.ShapeDtypeStruct((N, R * H), jnp.float32),
        ),
    )(p, ws, wd, brf)


def _lrelu(t):
    return jnp.where(t > 0, t, 0.01 * t)


def _final_body(p2_ref, h1_ref, h0_ref, wo0_ref, bo0_ref, wo1_ref, bo1_ref,
                wo2_ref, bo2_ref, out_ref):
    h2 = p2_ref[0] + p2_ref[1]
    t2 = jnp.dot(h2, wo2_ref[...], preferred_element_type=jnp.float32) + bo2_ref[...]
    t1 = jnp.dot(h1_ref[...], wo1_ref[...],
                 preferred_element_type=jnp.float32) + bo1_ref[...]
    t0 = jnp.dot(h0_ref[...], wo0_ref[...],
                 preferred_element_type=jnp.float32) + bo0_ref[...]
    out_ref[...] = _lrelu(t2) + _lrelu(t1) + _lrelu(t0)


def _final(p2, h1, h0, wo0, bo0, wo1, bo1, wo2, bo2):
    grid = N // BN
    wspec = pl.BlockSpec((H, OUT_DIM), lambda i: (0, 0))
    bspec = pl.BlockSpec((1, OUT_DIM), lambda i: (0, 0))
    return pl.pallas_call(
        _final_body,
        grid=(grid,),
        in_specs=[
            pl.BlockSpec((2, BN, H), lambda i: (0, i, 0)),
            pl.BlockSpec((BN, H), lambda i: (i, 0)),
            pl.BlockSpec((BN, H), lambda i: (i, 0)),
            wspec, bspec, wspec, bspec, wspec, bspec,
        ],
        out_specs=pl.BlockSpec((BN, OUT_DIM), lambda i: (i, 0)),
        out_shape=jax.ShapeDtypeStruct((N, OUT_DIM), jnp.float32),
    )(p2, h1, h0, wo0, bo0, wo1, bo1, wo2, bo2)


def _sc_layer_body(a_hbm, b_hbm, idxa_hbm, idxb_hbm, dst_hbm, w_hbm, out_hbm,
                   idxa_v, idxb_v, dst_v, w_v, a0, a1, b0, b1, o0, o1,
                   zbuf, acc, sga0, sga1, sgb0, sgb1, ss0, ss1):
    cid = lax.axis_index("c")
    sid = lax.axis_index("s")
    ng = jnp.where(cid == 0, GPT0, GPT1)
    gbase = jnp.where(cid == 0, sid * GPT0, 16 * GPT0 + sid * GPT1)

    sbase = jnp.minimum(gbase, G - GPTMAX)
    off = gbase - sbase
    pltpu.sync_copy(idxa_hbm.at[pl.ds(sbase, GPTMAX)], idxa_v)
    pltpu.sync_copy(idxb_hbm.at[pl.ds(sbase, GPTMAX)], idxb_v)
    pltpu.sync_copy(dst_hbm.at[pl.ds(sbase, GPTMAX)], dst_v)
    pltpu.sync_copy(w_hbm.at[pl.ds(sbase, GPTMAX)], w_v)

    def zb(i, c):
        zbuf[i, 0:16] = jnp.zeros((16,), jnp.float32)
        zbuf[i, 16:32] = jnp.zeros((16,), jnp.float32)
        return c

    lax.fori_loop(0, ZCH, zb, 0)
    rbase = sid * ROWS_PER_TILE
    for j in range(ROWS_PER_TILE // ZCH):
        pltpu.sync_copy(zbuf.at[pl.ds(0, ZCH)],
                        acc.at[pl.ds(rbase + j * ZCH, ZCH)])
    plsc.subcore_barrier()

    abufs = (a0, a1)
    bbufs = (b0, b1)
    obufs = (o0, o1)
    sgas = (sga0, sga1)
    sgbs = (sgb0, sgb1)
    sss = (ss0, ss1)

    for p in range(2):
        pltpu.async_copy(a_hbm.at[idxa_v.at[off + p]], abufs[p], sgas[p])
        pltpu.async_copy(b_hbm.at[idxb_v.at[off + p]], bbufs[p], sgbs[p])

    def pair(k2, c):
        for p in range(2):
            k = k2 * 2 + p
            ab, bb, ob = abufs[p], bbufs[p], obufs[p]
            pltpu.make_async_copy(a_hbm.at[idxa_v.at[off + k]], ab, sgas[p]).wait()
            pltpu.make_async_copy(b_hbm.at[idxb_v.at[off + k]], bb, sgbs[p]).wait()

            @pl.when(k2 > 0)
            def _():
                pltpu.make_async_copy(ob, acc.at[dst_v.at[off + k]], sss[p]).wait()

            def ebody(j, cc):
                wv16 = w_v[off + k, pl.ds(j * 16, 16)]
                for ll in range(16):
                    i = j * 16 + ll
                    wv = wv16[ll]
                    ob[i, 0:16] = (ab[i, 0:16] + bb[i, 0:16]) * wv
                    ob[i, 16:32] = (ab[i, 16:32] + bb[i, 16:32]) * wv
                return cc

            lax.fori_loop(0, GROUP // 16, ebody, 0)

            @pl.when(k + 2 < ng)
            def _():
                pltpu.async_copy(a_hbm.at[idxa_v.at[off + k + 2]], ab, sgas[p])
                pltpu.async_copy(b_hbm.at[idxb_v.at[off + k + 2]], bb, sgbs[p])

            pltpu.async_copy(ob, acc.at[dst_v.at[off + k]], sss[p], add=True)
        return c

    lax.fori_loop(0, ng // 2, pair, 0)
    for p in range(2):
        pltpu.make_async_copy(obufs[p], acc.at[dst_v.at[off + ng - 2 + p]],
                              sss[p]).wait()
    plsc.subcore_barrier()

    pltpu.sync_copy(acc.at[pl.ds(rbase, ROWS_PER_TILE)],
                    out_hbm.at[cid, pl.ds(rbase, ROWS_PER_TILE)])


def _sc_layer(a2d, b2d, idxa2, idxb2, dst2, w2):
    mesh = plsc.VectorSubcoreMesh(core_axis_name="c", subcore_axis_name="s")
    kern = functools.partial(
        pl.kernel,
        mesh=mesh,
        compiler_params=pltpu.CompilerParams(use_tc_tiling_on_sc=False),
        out_type=jax.ShapeDtypeStruct((2, NP, H), jnp.float32),
        scratch_types=[
            pltpu.VMEM((GPTMAX, GROUP), jnp.int32),
            pltpu.VMEM((GPTMAX, GROUP), jnp.int32),
            pltpu.VMEM((GPTMAX, GROUP), jnp.int32),
            pltpu.VMEM((GPTMAX, GROUP), jnp.float32),
            pltpu.VMEM((GROUP, H), jnp.float32),
            pltpu.VMEM((GROUP, H), jnp.float32),
            pltpu.VMEM((GROUP, H), jnp.float32),
            pltpu.VMEM((GROUP, H), jnp.float32),
            pltpu.VMEM((GROUP, H), jnp.float32),
            pltpu.VMEM((GROUP, H), jnp.float32),
            pltpu.VMEM((ZCH, H), jnp.float32),
            pltpu.VMEM_SHARED((NP, H), jnp.float32),
            pltpu.SemaphoreType.DMA,
            pltpu.SemaphoreType.DMA,
            pltpu.SemaphoreType.DMA,
            pltpu.SemaphoreType.DMA,
            pltpu.SemaphoreType.DMA,
            pltpu.SemaphoreType.DMA,
        ],
    )(_sc_layer_body)
    return kern(a2d, b2d, idxa2, idxb2, dst2, w2)


def kernel(x, edge_time, lambda_sym, beta, Wf, bf, Wr1, br1, Wr2, br2,
           Wo0, bo0, Wo1, bo1, Wo2, bo2, edge_index, edge_type):
    ws1 = jnp.transpose(Wr1[:, :H, :], (1, 0, 2)).reshape(H, R * H)
    wd1 = jnp.transpose(Wr1[:, H:, :], (1, 0, 2)).reshape(H, R * H)
    ws2 = jnp.transpose(Wr2[:, :H, :], (1, 0, 2)).reshape(H, R * H)
    wd2 = jnp.transpose(Wr2[:, H:, :], (1, 0, 2)).reshape(H, R * H)
    brf1 = br1.reshape(1, R * H)
    brf2 = br2.reshape(1, R * H)
    bfr = bf.reshape(1, H)
    bo0r = bo0.reshape(1, OUT_DIM)
    bo1r = bo1.reshape(1, OUT_DIM)
    bo2r = bo2.reshape(1, OUT_DIM)

    pad = EP - E
    et2 = jnp.pad(edge_time, (0, pad)).reshape(G, GROUP)
    src2 = jnp.pad(edge_index[0], (0, pad)).reshape(G, GROUP)
    dst2 = jnp.pad(edge_index[1], (0, pad)).reshape(G, GROUP)
    typ2 = jnp.pad(edge_type, (0, pad)).reshape(G, GROUP)

    w2, idxa2, idxb2 = _edge_prep(et2, src2, dst2, typ2, lambda_sym, beta)

    h0, a1, b1 = _embed(x, Wf, bfr, ws1, wd1, brf1)
    p1 = _sc_layer(a1.reshape(NT, H), b1.reshape(NT, H),
                   idxa2, idxb2, dst2, w2)[:, :N, :]
    h1, a2, b2 = _mid(p1, ws2, wd2, brf2)
    p2 = _sc_layer(a2.reshape(NT, H), b2.reshape(NT, H),
                   idxa2, idxb2, dst2, w2)[:, :N, :]
    out = _final(p2, h1, h0, Wo0, bo0r, Wo1, bo1r, Wo2, bo2r)
    return out
